# baseline (speedup 1.0000x reference)
import functools

import jax
import jax.numpy as jnp
from jax import lax
from jax.experimental import pallas as pl
from jax.experimental.pallas import tpu as pltpu
from jax.experimental.pallas import tpu_sc as plsc


def _emb_gather_sc(idx_flat, table):
    n_rows = idx_flat.shape[0]
    _, D = table.shape
    info = plsc.get_sparse_core_info()
    nc, ns = info.num_cores, info.num_subcores
    nw = nc * ns
    rows_per_w = n_rows // nw
    ch = min(rows_per_w, 64)
    n_chunks = rows_per_w // ch

    mesh = plsc.VectorSubcoreMesh(core_axis_name="c", subcore_axis_name="s")

    @functools.partial(
        pl.kernel,
        mesh=mesh,
        out_type=jax.ShapeDtypeStruct((n_rows, D), table.dtype),
        scratch_types=[
            pltpu.VMEM((ch,), jnp.int32),
            pltpu.VMEM((ch, D), table.dtype),
            pltpu.SemaphoreType.DMA,
        ],
    )
    def k(table_hbm, idx_hbm, out_hbm, idx_v, buf_v, sem):
        wid = lax.axis_index("s") * nc + lax.axis_index("c")
        base = wid * rows_per_w
        for c in range(n_chunks):
            off = base + c * ch
            pltpu.sync_copy(idx_hbm.at[pl.ds(off, ch)], idx_v)
            pltpu.async_copy(table_hbm.at[idx_v], buf_v, sem).wait()
            pltpu.sync_copy(buf_v, out_hbm.at[pl.ds(off, ch)])

    return k(table, idx_flat)


def kernel(xBT, embedding):
    if xBT.ndim == 1:
        xBT = xBT[None, :]
    B, T = xBT.shape
    V, D = embedding.shape
    idx = xBT.reshape(-1).astype(jnp.int32)
    table_i32 = lax.bitcast_convert_type(
        embedding.reshape(V, D // 2, 2), jnp.int32)
    out_i32 = _emb_gather_sc(idx, table_i32)
    out = lax.bitcast_convert_type(out_i32, jnp.bfloat16)
    return out.reshape(B, T, D)


# speedup vs baseline: 31.9676x; 31.9676x over previous
import functools

import jax
import jax.numpy as jnp
from jax import lax
from jax.experimental import pallas as pl
from jax.experimental.pallas import tpu as pltpu
from jax.experimental.pallas import tpu_sc as plsc

_LANES = 16


def _emb_gather_sc(rowA, sA, rowB, sB, table):
    n_pairs = rowA.shape[0]
    _, D = table.shape
    info = plsc.get_sparse_core_info()
    nc, ns = info.num_cores, info.num_subcores
    nw = nc * ns
    pairs_per_w = n_pairs // nw
    ch = 32
    n_chunks = pairs_per_w // ch
    groups = D // _LANES

    mesh = plsc.VectorSubcoreMesh(core_axis_name="c", subcore_axis_name="s")

    @functools.partial(
        pl.kernel,
        mesh=mesh,
        out_type=jax.ShapeDtypeStruct((2 * n_pairs, D), table.dtype),
        scratch_types=[
            pltpu.VMEM((ch,), jnp.int32),
            pltpu.VMEM((ch,), jnp.int32),
            pltpu.VMEM((ch, _LANES), jnp.int32),
            pltpu.VMEM((ch, _LANES), jnp.int32),
            pltpu.VMEM((ch, D), jnp.int32),
            pltpu.VMEM((ch, D), jnp.int32),
            pltpu.SemaphoreType.DMA,
            pltpu.SemaphoreType.DMA,
            pltpu.SemaphoreType.DMA,
        ],
    )
    def k(table_hbm, rowA_hbm, sA_hbm, rowB_hbm, sB_hbm, out_hbm,
          idxA_v, idxB_v, sa_v, sb_v, bufA, bufB, semI, semA, semB):
        tbl32 = table_hbm.bitcast(jnp.int32)
        out32 = out_hbm.bitcast(jnp.int32)
        wid = lax.axis_index("s") * nc + lax.axis_index("c")
        base = wid * pairs_per_w
        for c in range(n_chunks):
            off = base + c * ch
            cp1 = pltpu.async_copy(rowA_hbm.at[pl.ds(off, ch)], idxA_v, semI)
            cp2 = pltpu.async_copy(rowB_hbm.at[pl.ds(off, ch)], idxB_v, semI)
            cp3 = pltpu.async_copy(
                sA_hbm.at[pl.ds(off, ch), pl.ds(0, _LANES)], sa_v, semI)
            cp4 = pltpu.async_copy(
                sB_hbm.at[pl.ds(off, ch), pl.ds(0, _LANES)], sb_v, semI)
            cp1.wait(); cp2.wait(); cp3.wait(); cp4.wait()
            ga = pltpu.async_copy(tbl32.at[idxA_v], bufA, semA)
            gb = pltpu.async_copy(tbl32.at[idxB_v], bufB, semB)
            ga.wait()
            gb.wait()

            def blend_pair(p, _):
                sa = sa_v[p, pl.ds(0, _LANES)]
                sb = sb_v[p, pl.ds(0, _LANES)]
                for t in range(groups):
                    sl = pl.ds(t * _LANES, _LANES)
                    a = bufA[p, sl]
                    b = bufB[p, sl]
                    lo = lax.shift_right_logical(a, sa) & 0xFFFF
                    hi = lax.shift_left(lax.shift_right_logical(b, sb), 16)
                    bufA[p, sl] = lo | hi
                return 0

            lax.fori_loop(0, ch, blend_pair, 0)
            pltpu.sync_copy(bufA, out32.at[pl.ds(off, ch)])

    return k(table, rowA, sA, rowB, sB)


def kernel(xBT, embedding):
    if xBT.ndim == 1:
        xBT = xBT[None, :]
    B, T = xBT.shape
    _, D = embedding.shape
    idx = xBT.reshape(-1).astype(jnp.int32).reshape(-1, 2)
    ia, ib = idx[:, 0], idx[:, 1]
    n_pairs = ia.shape[0]
    rowA = lax.shift_right_logical(ia, 1)
    sA = jnp.broadcast_to(lax.shift_left(ia & 1, 4)[:, None],
                          (n_pairs, _LANES))
    rowB = lax.shift_right_logical(ib, 1)
    sB = jnp.broadcast_to(lax.shift_left(ib & 1, 4)[:, None],
                          (n_pairs, _LANES))
    out = _emb_gather_sc(rowA, sA, rowB, sB, embedding)
    return out.reshape(B, T, D)


# speedup vs baseline: 36.8999x; 1.1543x over previous
import functools

import jax
import jax.numpy as jnp
from jax import lax
from jax.experimental import pallas as pl
from jax.experimental.pallas import tpu as pltpu
from jax.experimental.pallas import tpu_sc as plsc

_LANES = 16


def _emb_gather_sc(rowA, sA, rowB, sB, table):
    n_pairs = rowA.shape[0]
    _, D = table.shape
    info = plsc.get_sparse_core_info()
    nc, ns = info.num_cores, info.num_subcores
    nw = nc * ns
    pairs_per_w = n_pairs // nw
    ch = 16
    n_chunks = pairs_per_w // ch
    groups = D // _LANES

    mesh = plsc.VectorSubcoreMesh(core_axis_name="c", subcore_axis_name="s")

    @functools.partial(
        pl.kernel,
        mesh=mesh,
        out_type=jax.ShapeDtypeStruct((2 * n_pairs, D), table.dtype),
        scratch_types=[
            pltpu.VMEM((pairs_per_w,), jnp.int32),
            pltpu.VMEM((pairs_per_w,), jnp.int32),
            pltpu.VMEM((pairs_per_w * _LANES,), jnp.int32),
            pltpu.VMEM((pairs_per_w * _LANES,), jnp.int32),
            pltpu.VMEM((2, ch, D), jnp.int32),
            pltpu.VMEM((2, ch, D), jnp.int32),
            pltpu.VMEM((2, ch, D), jnp.int32),
            pltpu.SemaphoreType.DMA,
            pltpu.SemaphoreType.DMA,
            pltpu.SemaphoreType.DMA,
            pltpu.SemaphoreType.DMA,
        ],
    )
    def k(table_hbm, rowA_hbm, sA_hbm, rowB_hbm, sB_hbm, out_hbm,
          idxA_v, idxB_v, sa_v, sb_v, bufA, bufB, bufO,
          semI, semG0, semG1, semO):
        tbl32 = table_hbm.bitcast(jnp.int32)
        out32 = out_hbm.bitcast(jnp.int32)
        wid = lax.axis_index("s") * nc + lax.axis_index("c")
        base = wid * pairs_per_w
        semG = (semG0, semG1)

        cps = [
            pltpu.async_copy(rowA_hbm.at[pl.ds(base, pairs_per_w)],
                             idxA_v, semI),
            pltpu.async_copy(rowB_hbm.at[pl.ds(base, pairs_per_w)],
                             idxB_v, semI),
            pltpu.async_copy(
                sA_hbm.at[pl.ds(base * _LANES, pairs_per_w * _LANES)],
                sa_v, semI),
            pltpu.async_copy(
                sB_hbm.at[pl.ds(base * _LANES, pairs_per_w * _LANES)],
                sb_v, semI),
        ]
        for cp in cps:
            cp.wait()

        def fire_gathers(c):
            s = c % 2
            ga = pltpu.async_copy(
                tbl32.at[idxA_v.at[pl.ds(c * ch, ch)]], bufA.at[s], semG[s])
            gb = pltpu.async_copy(
                tbl32.at[idxB_v.at[pl.ds(c * ch, ch)]], bufB.at[s], semG[s])
            return ga, gb

        inflight = {0: fire_gathers(0)}
        out_cps = {}
        for c in range(n_chunks):
            s = c % 2
            ga, gb = inflight.pop(c)
            ga.wait()
            gb.wait()
            if c + 1 < n_chunks:
                inflight[c + 1] = fire_gathers(c + 1)
            if c - 2 in out_cps:
                out_cps.pop(c - 2).wait()

            def blend_pair(p, _, c=c, s=s):
                sa = sa_v[pl.ds((c * ch + p) * _LANES, _LANES)]
                sb = sb_v[pl.ds((c * ch + p) * _LANES, _LANES)]
                for t in range(groups):
                    sl = pl.ds(t * _LANES, _LANES)
                    a = bufA[s, p, sl]
                    b = bufB[s, p, sl]
                    lo = lax.shift_right_logical(a, sa) & 0xFFFF
                    hi = lax.shift_left(lax.shift_right_logical(b, sb), 16)
                    bufO[s, p, sl] = lo | hi
                return 0

            lax.fori_loop(0, ch, blend_pair, 0)
            out_cps[c] = pltpu.async_copy(
                bufO.at[s], out32.at[pl.ds(base + c * ch, ch)], semO)
        for cp in out_cps.values():
            cp.wait()

    return k(table, rowA, sA, rowB, sB)


def kernel(xBT, embedding):
    if xBT.ndim == 1:
        xBT = xBT[None, :]
    B, T = xBT.shape
    _, D = embedding.shape
    idx = xBT.reshape(-1).astype(jnp.int32).reshape(-1, 2)
    ia, ib = idx[:, 0], idx[:, 1]
    n_pairs = ia.shape[0]
    rowA = lax.shift_right_logical(ia, 1)
    sA = jnp.broadcast_to(lax.shift_left(ia & 1, 4)[:, None],
                          (n_pairs, _LANES)).reshape(-1)
    rowB = lax.shift_right_logical(ib, 1)
    sB = jnp.broadcast_to(lax.shift_left(ib & 1, 4)[:, None],
                          (n_pairs, _LANES)).reshape(-1)
    out = _emb_gather_sc(rowA, sA, rowB, sB, embedding)
    return out.reshape(B, T, D)
